# Initial kernel scaffold; baseline (speedup 1.0000x reference)
import jax
import jax.numpy as jnp
from jax.experimental import pallas as pl


def kernel(vertices, faces, edges, face_masks, edge_masks, embed_vertex, embed_angle, embed_norm, embed_area, proj_W, proj_b, sage_proj_W, sage_proj_b, sage_Wl, sage_bl, sage_Wr, ln_gamma, ln_beta):
    raise NotImplementedError("write your pallas kernel here")


# speedup vs baseline: 2.2704x; 2.2704x over previous
import functools

import numpy as np
import jax
import jax.numpy as jnp
from jax import lax
from jax.experimental import pallas as pl
from jax.experimental.pallas import tpu as pltpu
from jax.experimental.pallas import tpu_sc as plsc

NBINS = 128
EDIM = 64
HD = 512
NV, NF, NE = 25000, 50000, 150000

BLK = 512
NFP = 50176
NFPM = NFP + BLK
NEP = 150016
FPT = NFP // 32
EPT = NEP // 16
MW = 640
WIN = 64
SUPER = 16 * WIN
NSC = (NFP + 2 * SUPER - 1) // (2 * SUPER)
NSUP = NFP // SUPER
PIECE0 = 512
LCAP = (EPT + PIECE0 - 1) // PIECE0 * PIECE0 + PIECE0
RND = 32
BIGD = 1 << 22

_cg = np.cos(np.arange(1, 128) * np.pi / 128.0)
_THRA = np.concatenate([[3.0, 1.5], _cg]).astype(np.float32)[None]
_THRB = np.concatenate([[1.5], _cg, [-3.0]]).astype(np.float32)[None]
_IOTA129 = np.arange(129, dtype=np.float32)[None]
_EYE16 = np.eye(16, dtype=np.float32)


def _sc_face_gather_body(verts_hbm, faces_hbm, fm_hbm, fct_hbm, vl, fl, ol):
    c = lax.axis_index("c")
    s = lax.axis_index("s")
    wid = c * 16 + s
    base = wid * FPT
    pltpu.sync_copy(verts_hbm, vl)
    pltpu.sync_copy(faces_hbm.at[pl.ds(base * 3, FPT * 3)], fl)
    pltpu.sync_copy(fm_hbm.at[pl.ds(base, FPT)], ol.at[pl.ds(9 * FPT, FPT)])

    zz = jnp.zeros((16,), jnp.float32)

    def step(k, carry):
        p0 = k * 16
        fid3 = (p0 + lax.iota(jnp.int32, 16)) * 3
        for v in range(3):
            vidx = plsc.load_gather(fl, [fid3 + v])
            vp = vidx * 3
            for cc in range(3):
                ol[pl.ds((3 * v + cc) * FPT + p0, 16)] = (
                    plsc.load_gather(vl, [vp + cc]))
        for r in range(10, 16):
            ol[pl.ds(r * FPT + p0, 16)] = zz
        return carry

    lax.fori_loop(0, FPT // 16, step, 0)
    for r in range(16):
        pltpu.sync_copy(ol.at[pl.ds(r * FPT, FPT)],
                        fct_hbm.at[pl.ds(r * NFP + base, FPT)])


def _sc_face_gather(verts_flat, faces_flat, fm_f):
    mesh = plsc.VectorSubcoreMesh(core_axis_name="c", subcore_axis_name="s")
    f = pl.kernel(
        _sc_face_gather_body,
        out_type=jax.ShapeDtypeStruct((16 * NFP,), jnp.float32),
        mesh=mesh,
        scratch_types=[
            pltpu.VMEM((NV * 3,), jnp.float32),
            pltpu.VMEM((FPT * 3,), jnp.int32),
            pltpu.VMEM((16 * FPT,), jnp.float32),
        ],
        compiler_params=pltpu.CompilerParams(needs_layout_passes=False),
    )
    return f(verts_flat, faces_flat, fm_f).reshape(16, NFP)


def _tc_face_embed_body(fct_ref, thra_ref, thrb_ref, ev_ref, ea_ref, en_ref,
                        er_ref, pw_ref, pb_ref, sw_ref, sb_ref, h_ref, m_ref):
    f = fct_ref[...]
    v0, v1, v2 = f[0:3], f[3:6], f[6:9]
    fm = f[9:10]
    eps = 1e-8

    def _cos(a, b, c):
        e1 = b - a
        e2 = c - a
        num = jnp.sum(e1 * e2, axis=0, keepdims=True)
        n1 = jnp.sqrt(jnp.sum(e1 * e1, axis=0, keepdims=True))
        n2 = jnp.sqrt(jnp.sum(e2 * e2, axis=0, keepdims=True))
        return num / (n1 * n2 + eps)

    cs = jnp.concatenate(
        [_cos(v0, v1, v2), _cos(v1, v2, v0), _cos(v2, v0, v1)], axis=0)
    u = v1 - v0
    w = v2 - v0
    crs = jnp.concatenate([
        u[1:2] * w[2:3] - u[2:3] * w[1:2],
        u[2:3] * w[0:1] - u[0:1] * w[2:3],
        u[0:1] * w[1:2] - u[1:2] * w[0:1],
    ], axis=0)
    cn = jnp.sqrt(jnp.sum(crs * crs, axis=0, keepdims=True))
    area = 0.5 * cn
    normal = crs / (cn + eps)

    def q(x, hi, lo):
        t = (x - lo) * (float(NBINS) / (hi - lo))
        return jnp.clip(jnp.floor(t), 0.0, float(NBINS - 1)) + 1.0

    bv = q(f[0:9], 1.0, 0.0) * fm
    bn = q(normal, 1.0, -1.0) * fm
    br = q(area, 0.5, 0.0) * fm
    cs = jnp.where(fm > 0.5, cs, 2.0)
    t16 = jnp.concatenate([bv, bn, br, cs], axis=0)
    eye = (lax.broadcasted_iota(jnp.int32, (16, 16), 0)
           == lax.broadcasted_iota(jnp.int32, (16, 16), 1)).astype(jnp.float32)
    tt = lax.dot_general(t16, eye, (((0,), (0,)), ((), ())),
                         preferred_element_type=jnp.float32,
                         precision=lax.Precision.HIGHEST)

    iota = lax.broadcasted_iota(
        jnp.int32, (1, NBINS + 1), 1).astype(jnp.float32)
    thra = thra_ref[...]
    thrb = thrb_ref[...]

    def onehot_eq(col):
        return (col == iota).astype(jnp.float32)

    pieces = []
    ev = ev_ref[...]
    for r in range(9):
        pieces.append(jnp.dot(onehot_eq(tt[:, r:r + 1]), ev,
                              preferred_element_type=jnp.float32, precision=lax.Precision.HIGHEST))
    ea = ea_ref[...]
    for a in range(3):
        col = tt[:, 13 + a:14 + a]
        oh = ((col <= thra).astype(jnp.float32)
              - (col <= thrb).astype(jnp.float32))
        pieces.append(jnp.dot(oh, ea, preferred_element_type=jnp.float32, precision=lax.Precision.HIGHEST))
    en = en_ref[...]
    for n in range(3):
        pieces.append(jnp.dot(onehot_eq(tt[:, 9 + n:10 + n]), en,
                              preferred_element_type=jnp.float32, precision=lax.Precision.HIGHEST))
    er = er_ref[...]
    pieces.append(jnp.dot(onehot_eq(tt[:, 12:13]), er,
                          preferred_element_type=jnp.float32, precision=lax.Precision.HIGHEST))
    feats = jnp.concatenate(pieces, axis=1)

    x = jax.nn.gelu(jnp.dot(feats, pw_ref[...],
                            preferred_element_type=jnp.float32, precision=lax.Precision.HIGHEST) + pb_ref[...])
    msg = jnp.maximum(
        jnp.dot(x, sw_ref[...], preferred_element_type=jnp.float32, precision=lax.Precision.HIGHEST)
        + sb_ref[...], 0.0)
    valid = (pl.program_id(0) < NFP // BLK).astype(jnp.float32)
    h_ref[...] = x * valid
    m_ref[...] = jnp.concatenate(
        [msg, jnp.ones((BLK, MW - HD), jnp.float32)], axis=1) * valid


def _tc_face_embed(fct, ev, ea, en, er, pw, pb, sw, sb):
    grid = (NFPM // BLK,)
    tbl = pl.BlockSpec((NBINS + 1, EDIM), lambda i: (0, 0))
    return pl.pallas_call(
        _tc_face_embed_body,
        grid=grid,
        in_specs=[
            pl.BlockSpec((16, BLK),
                         lambda i: (0, jnp.minimum(i, NFP // BLK - 1))),
            pl.BlockSpec((1, NBINS + 1), lambda i: (0, 0)),
            pl.BlockSpec((1, NBINS + 1), lambda i: (0, 0)),
            tbl, tbl, tbl, tbl,
            pl.BlockSpec((16 * EDIM, HD), lambda i: (0, 0)),
            pl.BlockSpec((1, HD), lambda i: (0, 0)),
            pl.BlockSpec((HD, HD), lambda i: (0, 0)),
            pl.BlockSpec((1, HD), lambda i: (0, 0)),
        ],
        out_specs=[
            pl.BlockSpec((BLK, HD), lambda i: (i, 0)),
            pl.BlockSpec((BLK, MW), lambda i: (i, 0)),
        ],
        out_shape=[
            jax.ShapeDtypeStruct((NFPM, HD), jnp.float32),
            jax.ShapeDtypeStruct((NFPM, MW), jnp.float32),
        ],
    )(fct, jnp.asarray(_THRA), jnp.asarray(_THRB), ev, ea, en, er, pw, pb,
      sw, sb)


PIECE = 512


def _sc_aggregate_body(m_hbm, src_hbm, dst_hbm, z_hbm,
                       agg_hbm,
                       src_l, dst_l, sel_s, sel_o, lb_s, lb_o, st_s, st_o,
                       roundb, slot, acc, cv, ixb, lsp_s, lsp_o, csp, sem):
    c = lax.axis_index("c")
    s = lax.axis_index("s")
    ebase = s * EPT
    pltpu.sync_copy(src_hbm.at[pl.ds(ebase, EPT)], src_l)
    pltpu.sync_copy(dst_hbm.at[pl.ds(ebase, EPT)], dst_l)

    zz = lax.iota(jnp.int32, 16) * 0
    lane0 = lax.iota(jnp.int32, 16) == 0

    def pass_body(p, pcarry):
        sc_idx = 2 * p + c
        lo = sc_idx * SUPER

        @pl.when(sc_idx < NSUP)
        def _phase1():
            def scan_step(k, cur):
                p0 = k * 16
                d = dst_l[pl.ds(p0, 16)]
                sv = src_l[pl.ds(p0, 16)]
                msk = (d >= lo) & (d < lo + SUPER)
                plsc.store_compressed(sel_s.at[pl.ds(cur, 16)], sv, mask=msk)
                plsc.store_compressed(sel_o.at[pl.ds(cur, 16)], d - lo,
                                      mask=msk)
                return cur + plsc.all_reduce_population_count(msk)[0]

            nsel = lax.fori_loop(0, EPT // 16, scan_step, jnp.int32(0))

            def pub_step(j, carry):
                pltpu.sync_copy(sel_s.at[pl.ds(j * PIECE, PIECE)],
                                lsp_s.at[pl.ds(s * LCAP + j * PIECE, PIECE)])
                pltpu.sync_copy(sel_o.at[pl.ds(j * PIECE, PIECE)],
                                lsp_o.at[pl.ds(s * LCAP + j * PIECE, PIECE)])
                return carry

            lax.fori_loop(0, (nsel + PIECE - 1) // PIECE, pub_step, 0)
            cv[pl.ds(0, 16)] = zz + nsel
            pltpu.sync_copy(cv, csp.at[pl.ds(s * 16, 16)])


        plsc.subcore_barrier()

        @pl.when(sc_idx < NSUP)
        def _phase2():
            for j in range(8):
                pltpu.sync_copy(z_hbm, acc.at[pl.ds(j * 8, 8)])
            for j in range(RND * WIN // 16):
                roundb[pl.ds(j * 16, 16)] = zz + NFP
            for j in range(WIN // 16):
                slot[pl.ds(j * 16, 16)] = zz
            wlo = s * WIN

            def flush_all():
                @pl.loop(0, RND)
                def _rnd(rr):
                    for j4 in range(WIN // 16):
                        ixb[pl.ds(j4 * 16, 16)] = (
                            roundb[pl.ds(rr * WIN + j4 * 16, 16)])
                    pltpu.async_copy(m_hbm.at[ixb], acc, sem,
                                     add=True).wait()
                for j2 in range(RND * WIN // 16):
                    roundb[pl.ds(j2 * 16, 16)] = zz + NFP
                for j2 in range(WIN // 16):
                    slot[pl.ds(j2 * 16, 16)] = zz

            def list_body(k, mrk):
                lst = (s + k) & 15
                pltpu.sync_copy(csp.at[pl.ds(lst * 16, 16)], cv)
                cl = cv[pl.ds(0, 16)][0]

                def piece_body(j, mr):
                    pltpu.sync_copy(
                        lsp_s.at[pl.ds(lst * LCAP + j * PIECE, PIECE)], lb_s)
                    pltpu.sync_copy(
                        lsp_o.at[pl.ds(lst * LCAP + j * PIECE, PIECE)], lb_o)

                    def it_body(i, mr2):
                        b0 = i * 16
                        gidx = j * PIECE + b0 + lax.iota(jnp.int32, 16)
                        ov = lb_o[pl.ds(b0, 16)]
                        sv = lb_s[pl.ds(b0, 16)]
                        msk = (gidx < cl) & (ov >= wlo) & (ov < wlo + WIN)
                        plsc.store_compressed(st_o.at[pl.ds(0, 16)],
                                              ov - wlo, mask=msk)
                        plsc.store_compressed(st_s.at[pl.ds(0, 16)], sv,
                                              mask=msk)
                        n = plsc.all_reduce_population_count(msk)[0]

                        def slot_step(e, mr3):
                            ev = zz + e
                            offv = plsc.load_gather(st_o, [ev])
                            valv = plsc.load_gather(st_s, [ev])
                            rv = plsc.load_gather(slot, [offv])

                            @pl.when(rv[0] >= RND)
                            def _fl():
                                flush_all()

                            r2v = plsc.load_gather(slot, [offv])
                            plsc.store_scatter(roundb, [r2v * WIN + offv],
                                               valv, mask=lane0)
                            plsc.store_scatter(slot, [offv], r2v + 1,
                                               mask=lane0)
                            return jnp.maximum(mr3, r2v[0] + 1)

                        return lax.fori_loop(0, n, slot_step, mr2)

                    return lax.fori_loop(0, PIECE // 16, it_body, mr)

                return lax.fori_loop(0, (cl + PIECE - 1) // PIECE,
                                     piece_body, mrk)

            maxr = lax.fori_loop(0, 16, list_body, jnp.int32(0))

            @pl.loop(0, RND)
            def _final(rr):
                for j4 in range(WIN // 16):
                    ixb[pl.ds(j4 * 16, 16)] = (
                        roundb[pl.ds(rr * WIN + j4 * 16, 16)])
                pltpu.async_copy(m_hbm.at[ixb], acc, sem, add=True).wait()

            pltpu.sync_copy(acc, agg_hbm.at[pl.ds(lo + wlo, WIN)])


        plsc.subcore_barrier()
        return pcarry

    lax.fori_loop(0, NSC, pass_body, 0)


def _sc_aggregate(m, srcp, dstp):
    mesh = plsc.VectorSubcoreMesh(core_axis_name="c", subcore_axis_name="s")
    z = jnp.zeros((8, MW), jnp.float32)
    f = pl.kernel(
        _sc_aggregate_body,
        out_type=jax.ShapeDtypeStruct((NFP, MW), jnp.float32),
        mesh=mesh,
        scratch_types=[
            pltpu.VMEM((EPT,), jnp.int32),
            pltpu.VMEM((EPT,), jnp.int32),
            pltpu.VMEM((LCAP,), jnp.int32),
            pltpu.VMEM((LCAP,), jnp.int32),
            pltpu.VMEM((PIECE,), jnp.int32),
            pltpu.VMEM((PIECE,), jnp.int32),
            pltpu.VMEM((16,), jnp.int32),
            pltpu.VMEM((16,), jnp.int32),
            pltpu.VMEM((RND * WIN,), jnp.int32),
            pltpu.VMEM((WIN,), jnp.int32),
            pltpu.VMEM((WIN, MW), jnp.float32),
            pltpu.VMEM((16,), jnp.int32),
            pltpu.VMEM((WIN,), jnp.int32),
            pltpu.VMEM_SHARED((16 * LCAP,), jnp.int32),
            pltpu.VMEM_SHARED((16 * LCAP,), jnp.int32),
            pltpu.VMEM_SHARED((256,), jnp.int32),
            pltpu.SemaphoreType.DMA,
        ],
        compiler_params=pltpu.CompilerParams(needs_layout_passes=False),
    )
    return f(m, srcp, dstp, z)


def _tc_finish_body(agg_ref, h_ref, fm_ref, wl_ref, bl_ref, wr_ref,
                    g_ref, b_ref, o_ref):
    aggw = agg_ref[...]
    agg = aggw[:, :HD]
    cnt = aggw[:, HD:HD + 1]
    h = h_ref[...]
    mean = agg / jnp.maximum(cnt, 1.0)
    out = (jnp.dot(mean, wl_ref[...], preferred_element_type=jnp.float32, precision=lax.Precision.HIGHEST)
           + bl_ref[...]
           + jnp.dot(h, wr_ref[...], preferred_element_type=jnp.float32, precision=lax.Precision.HIGHEST))
    nrm = jnp.sqrt(jnp.sum(out * out, axis=1, keepdims=True))
    out = out / jnp.maximum(nrm, 1e-12)
    out = jax.nn.gelu(out)
    mu = jnp.mean(out, axis=1, keepdims=True)
    var = jnp.mean((out - mu) ** 2, axis=1, keepdims=True)
    out = (out - mu) * lax.rsqrt(var + 1e-5) * g_ref[...] + b_ref[...]
    o_ref[...] = out * fm_ref[...]


def _tc_finish(agg, h, fm_col, wl, bl, wr, g, b):
    grid = (NFP // BLK,)
    return pl.pallas_call(
        _tc_finish_body,
        grid=grid,
        in_specs=[
            pl.BlockSpec((BLK, MW), lambda i: (i, 0)),
            pl.BlockSpec((BLK, HD), lambda i: (i, 0)),
            pl.BlockSpec((BLK, 1), lambda i: (i, 0)),
            pl.BlockSpec((HD, HD), lambda i: (0, 0)),
            pl.BlockSpec((1, HD), lambda i: (0, 0)),
            pl.BlockSpec((HD, HD), lambda i: (0, 0)),
            pl.BlockSpec((1, HD), lambda i: (0, 0)),
            pl.BlockSpec((1, HD), lambda i: (0, 0)),
        ],
        out_specs=pl.BlockSpec((BLK, HD), lambda i: (i, 0)),
        out_shape=jax.ShapeDtypeStruct((NFP, HD), jnp.float32),
    )(agg, h, fm_col, wl, bl, wr, g, b)


@jax.jit
def kernel(vertices, faces, edges, face_masks, edge_masks, embed_vertex,
           embed_angle, embed_norm, embed_area, proj_W, proj_b, sage_proj_W,
           sage_proj_b, sage_Wl, sage_bl, sage_Wr, ln_gamma, ln_beta):
    b = vertices.shape[0]
    nf = faces.shape[1]
    verts_flat = vertices.reshape(-1)
    faces_flat = jnp.pad(faces.reshape(-1).astype(jnp.int32),
                         (0, 3 * (NFP - nf)))
    fm_f = jnp.pad(face_masks.reshape(-1).astype(jnp.float32), (0, NFP - nf))

    src = edges[0, :, 0].astype(jnp.int32)
    dst = edges[0, :, 1].astype(jnp.int32)
    em = edge_masks.reshape(-1)
    dstm = jnp.where(em, dst, jnp.int32(BIGD))
    srcp = jnp.pad(src, (0, NEP - NE))
    dstp = jnp.pad(dstm, (0, NEP - NE), constant_values=BIGD)

    fct = _sc_face_gather(verts_flat, faces_flat, fm_f)
    h, m = _tc_face_embed(fct, embed_vertex, embed_angle, embed_norm,
                          embed_area, proj_W, proj_b.reshape(1, HD),
                          sage_proj_W, sage_proj_b.reshape(1, HD))
    sel = dstp < NFP
    w = sel.astype(jnp.float32)
    d = jnp.where(sel, dstp, NFP)
    agg = jax.ops.segment_sum(m[srcp] * w[:, None], d,
                              num_segments=NFP + 1)[:NFP]
    out = _tc_finish(agg, h, fm_f.reshape(NFP, 1), sage_Wl,
                     sage_bl.reshape(1, HD), sage_Wr, ln_gamma.reshape(1, HD),
                     ln_beta.reshape(1, HD))
    return out[:nf].reshape(b, nf, HD)
